# Initial kernel scaffold; baseline (speedup 1.0000x reference)
import jax
import jax.numpy as jnp
from jax.experimental import pallas as pl


def kernel(x):
    raise NotImplementedError("write your pallas kernel here")


# speedup vs baseline: 5.6955x; 5.6955x over previous
import jax
import jax.numpy as jnp
import numpy as np
from jax.experimental import pallas as pl

_PERM = tuple(int(i) for i in np.asarray(jax.random.permutation(jax.random.key(42), 8)))

_BB = 256


def _shuffle_block(x_ref, o_ref):
    for j, p in enumerate(_PERM):
        o_ref[:, j, :] = x_ref[:, p, :]


def kernel(x):
    n, s, d = x.shape
    return pl.pallas_call(
        _shuffle_block,
        grid=(n // _BB,),
        in_specs=[pl.BlockSpec((_BB, s, d), lambda i: (i, 0, 0))],
        out_specs=pl.BlockSpec((_BB, s, d), lambda i: (i, 0, 0)),
        out_shape=jax.ShapeDtypeStruct((n, s, d), x.dtype),
    )(x)
